# baseline (speedup 1.0000x reference)
import functools

import jax
import jax.numpy as jnp
from jax import lax
from jax.experimental import pallas as pl
from jax.experimental.pallas import tpu as pltpu
from jax.experimental.pallas import tpu_sc as plsc

N = 10000
D = 128
HID = 128
H = 8
C = 16
NG = 64
OUT = 16

NPAD = 10240
CHUNK = 128
NW = 32
ROWS_PER_TILE = NPAD // 16
WB = 5

f32 = jnp.float32
i32 = jnp.int32


def _split_write(ref, v):
    ref[0] = v[:, :64]
    ref[1] = v[:, 64:]


def _enc_body(x_ref, encw, encb, wl, bl, wr, br, xl_ref, xr_ref):
    x = x_ref[...]
    h = jnp.maximum(
        lax.dot_general(x, encw[...], (((1,), (1,)), ((), ())),
                        preferred_element_type=f32) + encb[...], 0.0)
    _split_write(xl_ref, lax.dot_general(h, wl[...], (((1,), (1,)), ((), ())),
                                         preferred_element_type=f32) + bl[...])
    _split_write(xr_ref, lax.dot_general(h, wr[...], (((1,), (1,)), ((), ())),
                                         preferred_element_type=f32) + br[...])


def _encode(x, encW, encb, wl, bl, wr, br):
    R = 1000
    grid = (N // R,)
    full = pl.BlockSpec((128, 128), lambda i: (0, 0))
    vec = pl.BlockSpec((1, 128), lambda i: (0, 0))
    row = pl.BlockSpec((R, 128), lambda i: (i, 0))
    srow = pl.BlockSpec((2, R, 64), lambda i: (0, i, 0))
    return pl.pallas_call(
        _enc_body,
        grid=grid,
        in_specs=[row, full, vec, full, vec, full, vec],
        out_specs=[srow, srow],
        out_shape=[jax.ShapeDtypeStruct((2, NPAD, 64), f32),
                   jax.ShapeDtypeStruct((2, NPAD, 64), f32)],
    )(x, encW, encb.reshape(1, 128), wl, bl.reshape(1, 128),
      wr, br.reshape(1, 128))


HHALF = 4


def _edge_kernel_body(xl_hbm, xr_hbm, src_hbm, dst_hbm, att_hbm,
                      num_out, den_out,
                      srcv0, srcv1, dstv0, dstv1, xj0, xj1, xi0, xi1,
                      wrow0, wrow1, drow0, drow1, attv,
                      ga0, ga1, gb0, gb1, sa0, sa1, sb0, sb1,
                      num_sh, den_sh):
    srcv_b, dstv_b = (srcv0, srcv1), (dstv0, dstv1)
    xj_b, xi_b = (xj0, xj1), (xi0, xi1)
    wrow_b, drow_b = (wrow0, wrow1), (drow0, drow1)
    ga_b, gb_b = (ga0, ga1), (gb0, gb1)
    sa_b, sb_b = (sa0, sa1), (sb0, sb1)
    xj, wrow, drow = xj0, wrow0, drow0
    cid = lax.axis_index("c")
    sid = lax.axis_index("s")
    lane = lax.broadcasted_iota(i32, (16,), 0)

    def _zrow(r, _):
        for h in range(HHALF):
            wrow[r, pl.ds(16 * h, 16)] = jnp.zeros((16,), f32)
        drow[r, :] = jnp.zeros((16,), f32)
        return 0
    lax.fori_loop(0, CHUNK, _zrow, 0)
    row0 = sid * ROWS_PER_TILE
    for k in range(WB):
        pltpu.sync_copy(wrow, num_sh.at[pl.ds(row0 + k * CHUNK, CHUNK)])
        pltpu.sync_copy(drow, den_sh.at[pl.ds(row0 + k * CHUNK, CHUNK)])
    pltpu.sync_copy(att_hbm.at[pl.ds(cid * 64, 64)], attv)
    plsc.subcore_barrier()

    ept = src_hbm.shape[0] // 16
    nchunks = ept // CHUNK

    att_regs = [attv[pl.ds(16 * h, 16)] for h in range(HHALF)]
    hmasks = [lane == h for h in range(HHALF)]
    perms = [(lane + sh) % 16 for sh in (8, 4, 2, 1)]

    def _hsum(v):
        for p in perms:
            v = v + v.at[p].get(mode='promise_in_bounds')
        return v

    off = cid * NPAD

    def _fire(ch, p):
        base = sid * ept + ch * CHUNK
        pltpu.sync_copy(src_hbm.at[pl.ds(base, CHUNK)], srcv_b[p])
        pltpu.sync_copy(dst_hbm.at[pl.ds(base, CHUNK)], dstv_b[p])
        for i in range(CHUNK // 16):
            srcv_b[p][pl.ds(16 * i, 16)] = (
                srcv_b[p][pl.ds(16 * i, 16)] + off)
        pltpu.async_copy(xl_hbm.at[srcv_b[p]], xj_b[p], ga_b[p])
        pltpu.async_copy(xr_hbm.at[dstv_b[p]], xi_b[p], gb_b[p])

    def _consume(ch, p):
        pltpu.make_async_copy(xl_hbm.at[srcv_b[p]], xj_b[p], ga_b[p]).wait()
        pltpu.make_async_copy(xr_hbm.at[dstv_b[p]], xi_b[p], gb_b[p]).wait()
        xjp, xip = xj_b[p], xi_b[p]
        wrp, drp = wrow_b[p], drow_b[p]

        @pl.when(ch >= 2)
        def _():
            pltpu.make_async_copy(wrp, num_sh.at[dstv_b[p]], sa_b[p]).wait()
            pltpu.make_async_copy(drp, den_sh.at[dstv_b[p]], sb_b[p]).wait()

        @plsc.parallel_loop(0, CHUNK, unroll=8)
        def _edge(e):
            z = jnp.zeros((16,), f32)
            for h in range(HHALF):
                xjh = xjp[e, pl.ds(16 * h, 16)]
                xih = xip[e, pl.ds(16 * h, 16)]
                v = xih + xjh
                lr = jnp.maximum(v, 0.2 * v)
                wv = jnp.exp(_hsum(att_regs[h] * lr))
                wrp[e, pl.ds(16 * h, 16)] = xjh * wv
                z = jnp.where(hmasks[h], wv, z)
            drp[e, :] = z

        pltpu.async_copy(wrp, num_sh.at[dstv_b[p]], sa_b[p], add=True)
        pltpu.async_copy(drp, den_sh.at[dstv_b[p]], sb_b[p], add=True)

    _fire(0, 0)

    def _pair(c2, _):
        ch = 2 * c2

        @pl.when(ch + 1 < nchunks)
        def _():
            _fire(ch + 1, 1)
        _consume(ch, 0)

        @pl.when(ch + 2 < nchunks)
        def _():
            _fire(ch + 2, 0)

        @pl.when(ch + 1 < nchunks)
        def _():
            _consume(ch + 1, 1)
        return 0
    lax.fori_loop(0, (nchunks + 1) // 2, _pair, 0)

    for p in (0, 1):
        pltpu.make_async_copy(wrow_b[p], num_sh.at[dstv_b[p]],
                              sa_b[p]).wait()
        pltpu.make_async_copy(drow_b[p], den_sh.at[dstv_b[p]],
                              sb_b[p]).wait()

    plsc.subcore_barrier()
    for k in range(WB):
        r = row0 + k * CHUNK
        pltpu.sync_copy(num_sh.at[pl.ds(r, CHUNK)], xj)
        pltpu.sync_copy(xj, num_out.at[cid, pl.ds(r, CHUNK)])
        pltpu.sync_copy(den_sh.at[pl.ds(r, CHUNK)], drow)
        pltpu.sync_copy(drow, den_out.at[cid, pl.ds(r, CHUNK)])


def _edge_phase(xlcat, xrcat, src, dst, att_flat):
    mesh = plsc.VectorSubcoreMesh(core_axis_name="c", subcore_axis_name="s")
    k = pl.kernel(
        _edge_kernel_body,
        out_type=[jax.ShapeDtypeStruct((2, NPAD, 64), f32),
                  jax.ShapeDtypeStruct((2, NPAD, 16), f32)],
        mesh=mesh,
        scratch_types=[
            pltpu.VMEM((CHUNK,), i32),
            pltpu.VMEM((CHUNK,), i32),
            pltpu.VMEM((CHUNK,), i32),
            pltpu.VMEM((CHUNK,), i32),
            pltpu.VMEM((CHUNK, 64), f32),
            pltpu.VMEM((CHUNK, 64), f32),
            pltpu.VMEM((CHUNK, 64), f32),
            pltpu.VMEM((CHUNK, 64), f32),
            pltpu.VMEM((CHUNK, 64), f32),
            pltpu.VMEM((CHUNK, 64), f32),
            pltpu.VMEM((CHUNK, 16), f32),
            pltpu.VMEM((CHUNK, 16), f32),
            pltpu.VMEM((64,), f32),
            pltpu.SemaphoreType.DMA,
            pltpu.SemaphoreType.DMA,
            pltpu.SemaphoreType.DMA,
            pltpu.SemaphoreType.DMA,
            pltpu.SemaphoreType.DMA,
            pltpu.SemaphoreType.DMA,
            pltpu.SemaphoreType.DMA,
            pltpu.SemaphoreType.DMA,
            pltpu.VMEM_SHARED((NPAD, 64), f32),
            pltpu.VMEM_SHARED((NPAD, 16), f32),
        ],
        compiler_params=pltpu.CompilerParams(use_tc_tiling_on_sc=False),
    )
    return k(xlcat, xrcat, src, dst, att_flat)


def _den_bcast(den):
    jj = lax.broadcasted_iota(i32, (32, 128), 0)
    kk = lax.broadcasted_iota(i32, (32, 128), 1)
    hh = kk // 16
    tgt = jnp.where(hh < 4, hh, hh + 12)
    erep = (jj == tgt).astype(f32)
    return lax.dot_general(den, erep, (((1,), (0,)), ((), ())),
                           preferred_element_type=f32)


def _mid_body(num_ref, den_ref, gb, lng, lnb, wl, bl, wr, br,
              xl_ref, xr_ref):
    num = jnp.concatenate([num_ref[0], num_ref[1]], axis=1)
    den = jnp.concatenate([den_ref[0], den_ref[1]], axis=1)
    denb = _den_bcast(den)
    h = jnp.maximum(num / (denb + 1e-16) + gb[...], 0.0)
    mu = jnp.mean(h, axis=1, keepdims=True)
    var = jnp.mean((h - mu) ** 2, axis=1, keepdims=True)
    hn = (h - mu) / jnp.sqrt(var + 1e-5) * lng[...] + lnb[...]
    _split_write(xl_ref, lax.dot_general(hn, wl[...], (((1,), (1,)), ((), ())),
                                         preferred_element_type=f32) + bl[...])
    _split_write(xr_ref, lax.dot_general(hn, wr[...], (((1,), (1,)), ((), ())),
                                         preferred_element_type=f32) + br[...])


def _mid(num, den, gbias, lng, lnb, wl, bl, wr, br):
    R = 1000
    grid = (N // R,)
    full = pl.BlockSpec((128, 128), lambda i: (0, 0))
    vec = pl.BlockSpec((1, 128), lambda i: (0, 0))
    nspec = pl.BlockSpec((2, R, 64), lambda i: (0, i, 0))
    dspec = pl.BlockSpec((2, R, 16), lambda i: (0, i, 0))
    srow = pl.BlockSpec((2, R, 64), lambda i: (0, i, 0))
    return pl.pallas_call(
        _mid_body,
        grid=grid,
        in_specs=[nspec, dspec, vec, vec, vec, full, vec, full, vec],
        out_specs=[srow, srow],
        out_shape=[jax.ShapeDtypeStruct((2, NPAD, 64), f32),
                   jax.ShapeDtypeStruct((2, NPAD, 64), f32)],
    )(num, den, gbias.reshape(1, 128), lng.reshape(1, 128),
      lnb.reshape(1, 128), wl, bl.reshape(1, 128), wr, br.reshape(1, 128))


def _head_body(num_ref, den_ref, gb, batch_ref, w1, b1, w2, b2,
               out_ref, sums, counts):
    i = pl.program_id(0)
    nsteps = pl.num_programs(0)

    @pl.when(i == 0)
    def _():
        sums[...] = jnp.zeros_like(sums)
        counts[...] = jnp.zeros_like(counts)

    num = jnp.concatenate([num_ref[0], num_ref[1]], axis=1)
    den = jnp.concatenate([den_ref[0], den_ref[1]], axis=1)
    denb = _den_bcast(den)
    h = jnp.maximum(num / (denb + 1e-16) + gb[...], 0.0)

    bb = jnp.broadcast_to(batch_ref[0], (NG, h.shape[0]))
    gg = lax.broadcasted_iota(i32, (NG, h.shape[0]), 0)
    mask = (bb == gg).astype(f32)
    sums[...] += lax.dot_general(mask, h, (((1,), (0,)), ((), ())),
                                 preferred_element_type=f32)
    counts[...] += jnp.broadcast_to(
        jnp.sum(mask, axis=1, keepdims=True), (NG, 128))

    @pl.when(i == nsteps - 1)
    def _():
        pooled = sums[...] / jnp.maximum(counts[...], 1.0)
        z = jnp.maximum(
            lax.dot_general(pooled, w1[...], (((1,), (1,)), ((), ())),
                            preferred_element_type=f32) + b1[...], 0.0)
        out_ref[...] = lax.dot_general(
            z, w2[...], (((1,), (1,)), ((), ())),
            preferred_element_type=f32) + b2[...]


def _head(num, den, gbias, batch, w1, b1, w2, b2):
    R = 1000
    grid = (N // R,)
    vec = pl.BlockSpec((1, 128), lambda i: (0, 0))
    nspec = pl.BlockSpec((2, R, 64), lambda i: (0, i, 0))
    dspec = pl.BlockSpec((2, R, 16), lambda i: (0, i, 0))
    bspec = pl.BlockSpec((1, 1, R), lambda i: (i, 0, 0))
    return pl.pallas_call(
        _head_body,
        grid=grid,
        in_specs=[nspec, dspec, vec, bspec,
                  pl.BlockSpec((NG, 128), lambda i: (0, 0)),
                  pl.BlockSpec((1, NG), lambda i: (0, 0)),
                  pl.BlockSpec((OUT, NG), lambda i: (0, 0)),
                  pl.BlockSpec((1, OUT), lambda i: (0, 0))],
        out_specs=pl.BlockSpec((NG, OUT), lambda i: (0, 0)),
        out_shape=jax.ShapeDtypeStruct((NG, OUT), f32),
        scratch_shapes=[pltpu.VMEM((NG, 128), f32),
                        pltpu.VMEM((NG, 128), f32)],
    )(num, den, gbias.reshape(1, 128), batch.reshape(N // R, 1, R),
      w1, b1.reshape(1, NG), w2, b2.reshape(1, OUT))


def kernel(x, edge_index, batch, enc_W, enc_b, g1_Wl, g1_bl, g1_Wr, g1_br,
           g1_att, g1_bias, g2_Wl, g2_bl, g2_Wr, g2_br, g2_att, g2_bias,
           ln_g, ln_b, h1_W, h1_b, h2_W, h2_b):
    E = edge_index.shape[1]
    etot = E + N
    ept = ((etot + NW * CHUNK - 1) // (NW * CHUNK)) * CHUNK
    etot_pad = ept * NW
    loops = jnp.arange(N, dtype=edge_index.dtype)
    padv = jnp.full((etot_pad - etot,), N, dtype=edge_index.dtype)
    src = jnp.concatenate([edge_index[0], loops, padv])
    dst = jnp.concatenate([edge_index[1], loops, padv])

    def flat(a):
        return a.reshape(2 * NPAD, 64)

    xl1, xr1 = _encode(x, enc_W, enc_b, g1_Wl, g1_bl, g1_Wr, g1_br)
    num1, den1 = _edge_phase(flat(xl1), flat(xr1), src, dst,
                             g1_att.reshape(128))

    xl2, xr2 = _mid(num1[:, :N], den1[:, :N], g1_bias, ln_g, ln_b,
                    g2_Wl, g2_bl, g2_Wr, g2_br)
    num2, den2 = _edge_phase(flat(xl2), flat(xr2), src, dst,
                             g2_att.reshape(128))

    return _head(num2[:, :N], den2[:, :N], g2_bias, batch,
                 h1_W, h1_b, h2_W, h2_b)


# speedup vs baseline: 1.4079x; 1.4079x over previous
import functools

import jax
import jax.numpy as jnp
from jax import lax
from jax.experimental import pallas as pl
from jax.experimental.pallas import tpu as pltpu
from jax.experimental.pallas import tpu_sc as plsc

N = 10000
D = 128
HID = 128
H = 8
C = 16
NG = 64
OUT = 16

NPAD = 10240
CHUNK = 128
NW = 32
ROWS_PER_TILE = NPAD // 16
WB = 5

f32 = jnp.float32
i32 = jnp.int32


def _split_write(ref, v):
    ref[0] = v[:, :64]
    ref[1] = v[:, 64:]


def _enc_body(x_ref, encw, encb, wl, bl, wr, br, xl_ref, xr_ref):
    x = x_ref[...]
    h = jnp.maximum(
        lax.dot_general(x, encw[...], (((1,), (1,)), ((), ())),
                        preferred_element_type=f32) + encb[...], 0.0)
    _split_write(xl_ref, lax.dot_general(h, wl[...], (((1,), (1,)), ((), ())),
                                         preferred_element_type=f32) + bl[...])
    _split_write(xr_ref, lax.dot_general(h, wr[...], (((1,), (1,)), ((), ())),
                                         preferred_element_type=f32) + br[...])


def _encode(x, encW, encb, wl, bl, wr, br):
    R = 1000
    grid = (N // R,)
    full = pl.BlockSpec((128, 128), lambda i: (0, 0))
    vec = pl.BlockSpec((1, 128), lambda i: (0, 0))
    row = pl.BlockSpec((R, 128), lambda i: (i, 0))
    srow = pl.BlockSpec((2, R, 64), lambda i: (0, i, 0))
    return pl.pallas_call(
        _enc_body,
        grid=grid,
        in_specs=[row, full, vec, full, vec, full, vec],
        out_specs=[srow, srow],
        out_shape=[jax.ShapeDtypeStruct((2, NPAD, 64), f32),
                   jax.ShapeDtypeStruct((2, NPAD, 64), f32)],
    )(x, encW, encb.reshape(1, 128), wl, bl.reshape(1, 128),
      wr, br.reshape(1, 128))


HHALF = 4


def _edge_kernel_body(xl_hbm, xr_hbm, src_hbm, dst_hbm, att_hbm,
                      num_out, den_out,
                      srcv0, srcv1, srcv2, srcv3, dstv0, dstv1, dstv2, dstv3,
                      xj0, xj1, xi0, xi1,
                      wrow0, wrow1, drow0, drow1, attv,
                      ia0, ia1, ia2, ia3, ib0, ib1, ib2, ib3,
                      ga0, ga1, gb0, gb1, sa0, sa1, sb0, sb1,
                      num_sh, den_sh):
    srcv_b, dstv_b = (srcv0, srcv1, srcv2, srcv3), (dstv0, dstv1, dstv2, dstv3)
    ia_b, ib_b = (ia0, ia1, ia2, ia3), (ib0, ib1, ib2, ib3)
    xj_b, xi_b = (xj0, xj1), (xi0, xi1)
    wrow_b, drow_b = (wrow0, wrow1), (drow0, drow1)
    ga_b, gb_b = (ga0, ga1), (gb0, gb1)
    sa_b, sb_b = (sa0, sa1), (sb0, sb1)
    xj, wrow, drow = xj0, wrow0, drow0
    cid = lax.axis_index("c")
    sid = lax.axis_index("s")
    lane = lax.broadcasted_iota(i32, (16,), 0)

    def _zrow(r, _):
        for h in range(HHALF):
            wrow[r, pl.ds(16 * h, 16)] = jnp.zeros((16,), f32)
        drow[r, :] = jnp.zeros((16,), f32)
        return 0
    lax.fori_loop(0, CHUNK, _zrow, 0)
    row0 = sid * ROWS_PER_TILE
    for k in range(WB):
        pltpu.sync_copy(wrow, num_sh.at[pl.ds(row0 + k * CHUNK, CHUNK)])
        pltpu.sync_copy(drow, den_sh.at[pl.ds(row0 + k * CHUNK, CHUNK)])
    pltpu.sync_copy(att_hbm.at[pl.ds(cid * 64, 64)], attv)
    plsc.subcore_barrier()

    ept = src_hbm.shape[0] // 16
    nchunks = ept // CHUNK

    att_regs = [attv[pl.ds(16 * h, 16)] for h in range(HHALF)]
    hmasks = [lane == h for h in range(HHALF)]
    perms = [(lane + sh) % 16 for sh in (8, 4, 2, 1)]

    def _hsum(v):
        for p in perms:
            v = v + v.at[p].get(mode='promise_in_bounds')
        return v

    off = cid * NPAD

    def _stage(ch, q):
        base = sid * ept + ch * CHUNK
        pltpu.async_copy(src_hbm.at[pl.ds(base, CHUNK)], srcv_b[q], ia_b[q])
        pltpu.async_copy(dst_hbm.at[pl.ds(base, CHUNK)], dstv_b[q], ib_b[q])

    def _fire(ch, q):
        base = sid * ept + ch * CHUNK
        p = q % 2
        pltpu.make_async_copy(src_hbm.at[pl.ds(base, CHUNK)], srcv_b[q],
                              ia_b[q]).wait()
        pltpu.make_async_copy(dst_hbm.at[pl.ds(base, CHUNK)], dstv_b[q],
                              ib_b[q]).wait()
        for i in range(CHUNK // 16):
            srcv_b[q][pl.ds(16 * i, 16)] = (
                srcv_b[q][pl.ds(16 * i, 16)] + off)
        pltpu.async_copy(xl_hbm.at[srcv_b[q]], xj_b[p], ga_b[p])
        pltpu.async_copy(xr_hbm.at[dstv_b[q]], xi_b[p], gb_b[p])

    def _wait_scatter(q):
        p = q % 2
        pltpu.make_async_copy(wrow_b[p], num_sh.at[dstv_b[q]],
                              sa_b[p]).wait()
        pltpu.make_async_copy(drow_b[p], den_sh.at[dstv_b[q]],
                              sb_b[p]).wait()

    def _consume(ch, q):
        p = q % 2
        pltpu.make_async_copy(xl_hbm.at[srcv_b[q]], xj_b[p], ga_b[p]).wait()
        pltpu.make_async_copy(xr_hbm.at[dstv_b[q]], xi_b[p], gb_b[p]).wait()
        xjp, xip = xj_b[p], xi_b[p]
        wrp, drp = wrow_b[p], drow_b[p]

        @plsc.parallel_loop(0, CHUNK, unroll=4)
        def _edge(e):
            z = jnp.zeros((16,), f32)
            for h in range(HHALF):
                xjh = xjp[e, pl.ds(16 * h, 16)]
                xih = xip[e, pl.ds(16 * h, 16)]
                v = xih + xjh
                lr = jnp.maximum(v, 0.2 * v)
                wv = jnp.exp(_hsum(att_regs[h] * lr))
                wrp[e, pl.ds(16 * h, 16)] = xjh * wv
                z = jnp.where(hmasks[h], wv, z)
            drp[e, :] = z

        pltpu.async_copy(wrp, num_sh.at[dstv_b[q]], sa_b[p], add=True)
        pltpu.async_copy(drp, den_sh.at[dstv_b[q]], sb_b[p], add=True)

    _stage(0, 0)
    _stage(1, 1)
    _fire(0, 0)

    ngroups = (nchunks + 2 + 3) // 4

    def _group(g, _):
        ch0 = 4 * g
        for j in range(4):
            ch = ch0 + j

            @pl.when(jnp.logical_and(ch >= 2, ch < nchunks + 2))
            def _():
                _wait_scatter((j + 2) % 4)

            @pl.when(ch + 2 < nchunks)
            def _():
                _stage(ch + 2, (j + 2) % 4)

            @pl.when(ch + 1 < nchunks)
            def _():
                _fire(ch + 1, (j + 1) % 4)

            @pl.when(ch < nchunks)
            def _():
                _consume(ch, j)
        return 0
    lax.fori_loop(0, ngroups, _group, 0)

    plsc.subcore_barrier()
    for k in range(WB):
        r = row0 + k * CHUNK
        pltpu.sync_copy(num_sh.at[pl.ds(r, CHUNK)], xj)
        pltpu.sync_copy(xj, num_out.at[cid, pl.ds(r, CHUNK)])
        pltpu.sync_copy(den_sh.at[pl.ds(r, CHUNK)], drow)
        pltpu.sync_copy(drow, den_out.at[cid, pl.ds(r, CHUNK)])


def _edge_phase(xlcat, xrcat, src, dst, att_flat):
    mesh = plsc.VectorSubcoreMesh(core_axis_name="c", subcore_axis_name="s")
    k = pl.kernel(
        _edge_kernel_body,
        out_type=[jax.ShapeDtypeStruct((2, NPAD, 64), f32),
                  jax.ShapeDtypeStruct((2, NPAD, 16), f32)],
        mesh=mesh,
        scratch_types=[
            pltpu.VMEM((CHUNK,), i32),
            pltpu.VMEM((CHUNK,), i32),
            pltpu.VMEM((CHUNK,), i32),
            pltpu.VMEM((CHUNK,), i32),
            pltpu.VMEM((CHUNK,), i32),
            pltpu.VMEM((CHUNK,), i32),
            pltpu.VMEM((CHUNK,), i32),
            pltpu.VMEM((CHUNK,), i32),
            pltpu.VMEM((CHUNK, 64), f32),
            pltpu.VMEM((CHUNK, 64), f32),
            pltpu.VMEM((CHUNK, 64), f32),
            pltpu.VMEM((CHUNK, 64), f32),
            pltpu.VMEM((CHUNK, 64), f32),
            pltpu.VMEM((CHUNK, 64), f32),
            pltpu.VMEM((CHUNK, 16), f32),
            pltpu.VMEM((CHUNK, 16), f32),
            pltpu.VMEM((64,), f32),
        ] + [pltpu.SemaphoreType.DMA] * 16 + [
            pltpu.VMEM_SHARED((NPAD, 64), f32),
            pltpu.VMEM_SHARED((NPAD, 16), f32),
        ],
        compiler_params=pltpu.CompilerParams(use_tc_tiling_on_sc=False),
    )
    return k(xlcat, xrcat, src, dst, att_flat)


def _den_bcast(den):
    jj = lax.broadcasted_iota(i32, (32, 128), 0)
    kk = lax.broadcasted_iota(i32, (32, 128), 1)
    hh = kk // 16
    tgt = jnp.where(hh < 4, hh, hh + 12)
    erep = (jj == tgt).astype(f32)
    return lax.dot_general(den, erep, (((1,), (0,)), ((), ())),
                           preferred_element_type=f32)


def _mid_body(num_ref, den_ref, gb, lng, lnb, wl, bl, wr, br,
              xl_ref, xr_ref):
    num = jnp.concatenate([num_ref[0], num_ref[1]], axis=1)
    den = jnp.concatenate([den_ref[0], den_ref[1]], axis=1)
    denb = _den_bcast(den)
    h = jnp.maximum(num / (denb + 1e-16) + gb[...], 0.0)
    mu = jnp.mean(h, axis=1, keepdims=True)
    var = jnp.mean((h - mu) ** 2, axis=1, keepdims=True)
    hn = (h - mu) / jnp.sqrt(var + 1e-5) * lng[...] + lnb[...]
    _split_write(xl_ref, lax.dot_general(hn, wl[...], (((1,), (1,)), ((), ())),
                                         preferred_element_type=f32) + bl[...])
    _split_write(xr_ref, lax.dot_general(hn, wr[...], (((1,), (1,)), ((), ())),
                                         preferred_element_type=f32) + br[...])


def _mid(num, den, gbias, lng, lnb, wl, bl, wr, br):
    R = 1000
    grid = (N // R,)
    full = pl.BlockSpec((128, 128), lambda i: (0, 0))
    vec = pl.BlockSpec((1, 128), lambda i: (0, 0))
    nspec = pl.BlockSpec((2, R, 64), lambda i: (0, i, 0))
    dspec = pl.BlockSpec((2, R, 16), lambda i: (0, i, 0))
    srow = pl.BlockSpec((2, R, 64), lambda i: (0, i, 0))
    return pl.pallas_call(
        _mid_body,
        grid=grid,
        in_specs=[nspec, dspec, vec, vec, vec, full, vec, full, vec],
        out_specs=[srow, srow],
        out_shape=[jax.ShapeDtypeStruct((2, NPAD, 64), f32),
                   jax.ShapeDtypeStruct((2, NPAD, 64), f32)],
    )(num, den, gbias.reshape(1, 128), lng.reshape(1, 128),
      lnb.reshape(1, 128), wl, bl.reshape(1, 128), wr, br.reshape(1, 128))


def _head_body(num_ref, den_ref, gb, batch_ref, w1, b1, w2, b2,
               out_ref, sums, counts):
    i = pl.program_id(0)
    nsteps = pl.num_programs(0)

    @pl.when(i == 0)
    def _():
        sums[...] = jnp.zeros_like(sums)
        counts[...] = jnp.zeros_like(counts)

    num = jnp.concatenate([num_ref[0], num_ref[1]], axis=1)
    den = jnp.concatenate([den_ref[0], den_ref[1]], axis=1)
    denb = _den_bcast(den)
    h = jnp.maximum(num / (denb + 1e-16) + gb[...], 0.0)

    bb = jnp.broadcast_to(batch_ref[0], (NG, h.shape[0]))
    gg = lax.broadcasted_iota(i32, (NG, h.shape[0]), 0)
    mask = (bb == gg).astype(f32)
    sums[...] += lax.dot_general(mask, h, (((1,), (0,)), ((), ())),
                                 preferred_element_type=f32)
    counts[...] += jnp.broadcast_to(
        jnp.sum(mask, axis=1, keepdims=True), (NG, 128))

    @pl.when(i == nsteps - 1)
    def _():
        pooled = sums[...] / jnp.maximum(counts[...], 1.0)
        z = jnp.maximum(
            lax.dot_general(pooled, w1[...], (((1,), (1,)), ((), ())),
                            preferred_element_type=f32) + b1[...], 0.0)
        out_ref[...] = lax.dot_general(
            z, w2[...], (((1,), (1,)), ((), ())),
            preferred_element_type=f32) + b2[...]


def _head(num, den, gbias, batch, w1, b1, w2, b2):
    R = 1000
    grid = (N // R,)
    vec = pl.BlockSpec((1, 128), lambda i: (0, 0))
    nspec = pl.BlockSpec((2, R, 64), lambda i: (0, i, 0))
    dspec = pl.BlockSpec((2, R, 16), lambda i: (0, i, 0))
    bspec = pl.BlockSpec((1, 1, R), lambda i: (i, 0, 0))
    return pl.pallas_call(
        _head_body,
        grid=grid,
        in_specs=[nspec, dspec, vec, bspec,
                  pl.BlockSpec((NG, 128), lambda i: (0, 0)),
                  pl.BlockSpec((1, NG), lambda i: (0, 0)),
                  pl.BlockSpec((OUT, NG), lambda i: (0, 0)),
                  pl.BlockSpec((1, OUT), lambda i: (0, 0))],
        out_specs=pl.BlockSpec((NG, OUT), lambda i: (0, 0)),
        out_shape=jax.ShapeDtypeStruct((NG, OUT), f32),
        scratch_shapes=[pltpu.VMEM((NG, 128), f32),
                        pltpu.VMEM((NG, 128), f32)],
    )(num, den, gbias.reshape(1, 128), batch.reshape(N // R, 1, R),
      w1, b1.reshape(1, NG), w2, b2.reshape(1, OUT))


def kernel(x, edge_index, batch, enc_W, enc_b, g1_Wl, g1_bl, g1_Wr, g1_br,
           g1_att, g1_bias, g2_Wl, g2_bl, g2_Wr, g2_br, g2_att, g2_bias,
           ln_g, ln_b, h1_W, h1_b, h2_W, h2_b):
    E = edge_index.shape[1]
    etot = E + N
    ept = ((etot + NW * CHUNK - 1) // (NW * CHUNK)) * CHUNK
    etot_pad = ept * NW
    loops = jnp.arange(N, dtype=edge_index.dtype)
    padv = jnp.full((etot_pad - etot,), N, dtype=edge_index.dtype)
    src = jnp.concatenate([edge_index[0], loops, padv])
    dst = jnp.concatenate([edge_index[1], loops, padv])

    def flat(a):
        return a.reshape(2 * NPAD, 64)

    xl1, xr1 = _encode(x, enc_W, enc_b, g1_Wl, g1_bl, g1_Wr, g1_br)
    num1, den1 = _edge_phase(flat(xl1), flat(xr1), src, dst,
                             g1_att.reshape(128))

    xl2, xr2 = _mid(num1[:, :N], den1[:, :N], g1_bias, ln_g, ln_b,
                    g2_Wl, g2_bl, g2_Wr, g2_br)
    num2, den2 = _edge_phase(flat(xl2), flat(xr2), src, dst,
                             g2_att.reshape(128))

    return _head(num2[:, :N], den2[:, :N], g2_bias, batch,
                 h1_W, h1_b, h2_W, h2_b)
